# Initial kernel scaffold; baseline (speedup 1.0000x reference)
import jax
import jax.numpy as jnp
from jax.experimental import pallas as pl


def kernel(x, edge_index, W_gcn, b_gcn, W_diff, b_diff):
    raise NotImplementedError("write your pallas kernel here")


# speedup vs baseline: 10.3299x; 10.3299x over previous
import functools

import jax
import jax.numpy as jnp
from jax import lax
from jax.experimental import pallas as pl
from jax.experimental.pallas import tpu as pltpu
from jax.experimental.pallas import tpu_sc as plsc

NC = 2
NS = 16
L = 16


def _degree_body(np_, ept, ei_ref, hist_ref, idx_v, hist_v):
    c = lax.axis_index("c")
    s = lax.axis_index("s")
    row = 1 - c
    pltpu.sync_copy(ei_ref.at[row, pl.ds(s * ept, ept)], idx_v)
    zeros = jnp.zeros((L,), jnp.float32)
    ones = jnp.ones((L,), jnp.float32)

    def zero_body(i, _):
        hist_v[pl.ds(i * L, L)] = zeros
        return 0

    lax.fori_loop(0, np_ // L, zero_body, 0)

    def count_body(i, _):
        iv = idx_v[pl.ds(i * L, L)]
        plsc.addupdate_scatter(hist_v, [iv], ones)
        return 0

    lax.fori_loop(0, ept // L, count_body, 0)
    pltpu.sync_copy(hist_v, hist_ref.at[c, s])


def _gs_body(np_, cpt, sg, rpt, ei2_ref, z_ref, x_ref, zrow_ref, ts_ref,
             idx_g, idx_s, rows, acc):
    c = lax.axis_index("c")
    s = lax.axis_index("s")
    pltpu.sync_copy(zrow_ref.at[pl.ds(s * rpt, rpt)], acc.at[pl.ds(s * rpt, rpt)])
    plsc.subcore_barrier()

    def run(table_ref):
        def stage(st, _):
            base = s * cpt + st * sg
            pltpu.sync_copy(ei2_ref.at[c, pl.ds(base, sg)], idx_g)
            pltpu.sync_copy(ei2_ref.at[1 - c, pl.ds(base, sg)], idx_s)

            def body(j, _):
                pltpu.sync_copy(table_ref.at[idx_g.at[j]], rows)
                pltpu.sync_copy(rows, acc.at[idx_s.at[j]], add=True)
                return 0

            lax.fori_loop(0, sg, body, 0)
            return 0

        lax.fori_loop(0, cpt // sg, stage, 0)

    @pl.when(c == 0)
    def _():
        run(z_ref)

    @pl.when(c == 1)
    def _():
        run(x_ref)

    plsc.subcore_barrier()
    pltpu.sync_copy(acc.at[pl.ds(s * rpt, rpt)], ts_ref.at[c, pl.ds(s * rpt, rpt)])


def _z_body(x_ref, w_ref, h0_ref, z_ref):
    indeg = jnp.sum(h0_ref[...], axis=0)
    dis = lax.rsqrt(indeg + 1.0)
    xw = jnp.dot(x_ref[...], w_ref[...], preferred_element_type=jnp.float32)
    z_ref[...] = xw * dis[:, None]


def _final_body(x_ref, s_ref, t_ref, z_ref, h0_ref, h1_ref, wd_ref,
                bg_ref, bd_ref, o_ref):
    indeg = jnp.sum(h0_ref[...], axis=0)
    cnt = jnp.sum(h1_ref[...], axis=0)
    dis = lax.rsqrt(indeg + 1.0)
    h = jnp.maximum(dis[:, None] * (t_ref[...] + z_ref[...]) + bg_ref[...], 0.0)
    proj = jnp.dot(x_ref[...] * s_ref[...], wd_ref[...],
                   preferred_element_type=jnp.float32)
    o_ref[...] = h + proj + cnt[:, None] * bd_ref[...]


def kernel(x, edge_index, W_gcn, b_gcn, W_diff, b_diff):
    N, D = x.shape
    E = edge_index.shape[1]

    chunk = 128
    n_chunks = -(-E // (NS * 8 * chunk)) * (NS * 8)
    EP = n_chunks * chunk
    cpt = n_chunks // NS
    ept = EP // NS
    NP = -(-(N + 1) // (NS * 128)) * (NS * 128)
    rpt = NP // NS

    ei = jnp.concatenate(
        [edge_index.astype(jnp.int32),
         jnp.full((2, EP - E), N, jnp.int32)], axis=1)
    ei2 = ei.reshape(2, n_chunks, chunk)
    x_p = jnp.concatenate([x, jnp.zeros((NP - N, D), x.dtype)], axis=0)
    zrow = jnp.zeros((NP, D), jnp.float32)

    mesh = plsc.VectorSubcoreMesh(core_axis_name="c", subcore_axis_name="s")
    sc_params = pltpu.CompilerParams(needs_layout_passes=False)

    hist = pl.kernel(
        functools.partial(_degree_body, NP, ept),
        out_type=jax.ShapeDtypeStruct((NC, NS, NP), jnp.float32),
        mesh=mesh,
        compiler_params=sc_params,
        scratch_types=[
            pltpu.VMEM((ept,), jnp.int32),
            pltpu.VMEM((NP,), jnp.float32),
        ],
    )(ei)
    h0 = hist[0]
    h1 = hist[1]

    RB = rpt
    grid = NP // RB
    z = pl.pallas_call(
        _z_body,
        grid=(grid,),
        in_specs=[
            pl.BlockSpec((RB, D), lambda i: (i, 0)),
            pl.BlockSpec((D, D), lambda i: (0, 0)),
            pl.BlockSpec((NS, RB), lambda i: (0, i)),
        ],
        out_specs=pl.BlockSpec((RB, D), lambda i: (i, 0)),
        out_shape=jax.ShapeDtypeStruct((NP, D), jnp.float32),
    )(x_p, W_gcn, h0)

    sg = 32
    ts = pl.kernel(
        functools.partial(_gs_body, NP, cpt, sg, rpt),
        out_type=jax.ShapeDtypeStruct((NC, NP, D), jnp.float32),
        mesh=mesh,
        compiler_params=sc_params,
        scratch_types=[
            pltpu.VMEM((sg, chunk), jnp.int32),
            pltpu.VMEM((sg, chunk), jnp.int32),
            pltpu.VMEM((chunk, D), jnp.float32),
            pltpu.VMEM_SHARED((NP, D), jnp.float32),
        ],
    )(ei2, z, x_p, zrow)

    out = pl.pallas_call(
        _final_body,
        grid=(grid,),
        in_specs=[
            pl.BlockSpec((RB, D), lambda i: (i, 0)),
            pl.BlockSpec((RB, D), lambda i: (i, 0)),
            pl.BlockSpec((RB, D), lambda i: (i, 0)),
            pl.BlockSpec((RB, D), lambda i: (i, 0)),
            pl.BlockSpec((NS, RB), lambda i: (0, i)),
            pl.BlockSpec((NS, RB), lambda i: (0, i)),
            pl.BlockSpec((D, D), lambda i: (0, 0)),
            pl.BlockSpec((1, D), lambda i: (0, 0)),
            pl.BlockSpec((1, D), lambda i: (0, 0)),
        ],
        out_specs=pl.BlockSpec((RB, D), lambda i: (i, 0)),
        out_shape=jax.ShapeDtypeStruct((NP, D), jnp.float32),
    )(x_p, ts[1], ts[0], z, h0, h1, W_diff,
      b_gcn.reshape(1, D), b_diff.reshape(1, D))

    return out[:N]


# speedup vs baseline: 11.3479x; 1.0985x over previous
import functools

import jax
import jax.numpy as jnp
from jax import lax
from jax.experimental import pallas as pl
from jax.experimental.pallas import tpu as pltpu
from jax.experimental.pallas import tpu_sc as plsc

NC = 2
NS = 16
L = 16


def _degree_body(np_, ept, ei_ref, hist_ref, idx_v, hist_v):
    c = lax.axis_index("c")
    s = lax.axis_index("s")
    row = 1 - c
    pltpu.sync_copy(ei_ref.at[row, pl.ds(s * ept, ept)], idx_v)
    zeros = jnp.zeros((L,), jnp.float32)
    ones = jnp.ones((L,), jnp.float32)

    def zero_body(i, _):
        hist_v[pl.ds(i * L, L)] = zeros
        return 0

    lax.fori_loop(0, np_ // L, zero_body, 0)

    def count_body(i, _):
        iv = idx_v[pl.ds(i * L, L)]
        plsc.addupdate_scatter(hist_v, [iv], ones)
        return 0

    lax.fori_loop(0, ept // L, count_body, 0)
    pltpu.sync_copy(hist_v, hist_ref.at[c, s])


def _gs_body(np_, cpt, sg, rpt, ei2_ref, z_ref, x_ref, zrow_ref, ts_ref,
             idx_g, idx_s, rows0, rows1, acc, sem_g0, sem_g1, sem_s0, sem_s1):
    c = lax.axis_index("c")
    s = lax.axis_index("s")
    pltpu.sync_copy(zrow_ref.at[pl.ds(s * rpt, rpt)], acc.at[pl.ds(s * rpt, rpt)])
    plsc.subcore_barrier()

    def run(table_ref):
        def wait_gather(buf, sem):
            pltpu.make_async_copy(table_ref.at[idx_g.at[0]], buf, sem).wait()

        def wait_scatter(buf, sem):
            pltpu.make_async_copy(buf, acc.at[idx_s.at[0]], sem).wait()

        def stage(st, _):
            base = s * cpt + st * sg
            pltpu.sync_copy(ei2_ref.at[c, pl.ds(base, sg)], idx_g)
            pltpu.sync_copy(ei2_ref.at[1 - c, pl.ds(base, sg)], idx_s)
            pltpu.async_copy(table_ref.at[idx_g.at[0]], rows0, sem_g0)
            pltpu.async_copy(table_ref.at[idx_g.at[1]], rows1, sem_g1)

            def pair(p, _):
                j = 2 * p
                wait_gather(rows0, sem_g0)
                pltpu.async_copy(rows0, acc.at[idx_s.at[j]], sem_s0, add=True)
                wait_gather(rows1, sem_g1)
                pltpu.async_copy(rows1, acc.at[idx_s.at[j + 1]], sem_s1, add=True)
                wait_scatter(rows0, sem_s0)
                pltpu.async_copy(table_ref.at[idx_g.at[j + 2]], rows0, sem_g0)
                wait_scatter(rows1, sem_s1)
                pltpu.async_copy(table_ref.at[idx_g.at[j + 3]], rows1, sem_g1)
                return 0

            lax.fori_loop(0, sg // 2 - 1, pair, 0)
            j = sg - 2
            wait_gather(rows0, sem_g0)
            pltpu.async_copy(rows0, acc.at[idx_s.at[j]], sem_s0, add=True)
            wait_gather(rows1, sem_g1)
            pltpu.async_copy(rows1, acc.at[idx_s.at[j + 1]], sem_s1, add=True)
            wait_scatter(rows0, sem_s0)
            wait_scatter(rows1, sem_s1)
            return 0

        lax.fori_loop(0, cpt // sg, stage, 0)

    @pl.when(c == 0)
    def _():
        run(z_ref)

    @pl.when(c == 1)
    def _():
        run(x_ref)

    plsc.subcore_barrier()
    pltpu.sync_copy(acc.at[pl.ds(s * rpt, rpt)], ts_ref.at[c, pl.ds(s * rpt, rpt)])


def _z_body(x_ref, w_ref, h0_ref, z_ref):
    indeg = jnp.sum(h0_ref[...], axis=0)
    dis = lax.rsqrt(indeg + 1.0)
    xw = jnp.dot(x_ref[...], w_ref[...], preferred_element_type=jnp.float32)
    z_ref[...] = xw * dis[:, None]


def _final_body(x_ref, s_ref, t_ref, z_ref, h0_ref, h1_ref, wd_ref,
                bg_ref, bd_ref, o_ref):
    indeg = jnp.sum(h0_ref[...], axis=0)
    cnt = jnp.sum(h1_ref[...], axis=0)
    dis = lax.rsqrt(indeg + 1.0)
    h = jnp.maximum(dis[:, None] * (t_ref[...] + z_ref[...]) + bg_ref[...], 0.0)
    proj = jnp.dot(x_ref[...] * s_ref[...], wd_ref[...],
                   preferred_element_type=jnp.float32)
    o_ref[...] = h + proj + cnt[:, None] * bd_ref[...]


def kernel(x, edge_index, W_gcn, b_gcn, W_diff, b_diff):
    N, D = x.shape
    E = edge_index.shape[1]

    chunk = 128
    n_chunks = -(-E // (NS * 8 * chunk)) * (NS * 8)
    EP = n_chunks * chunk
    cpt = n_chunks // NS
    ept = EP // NS
    NP = -(-(N + 1) // (NS * 128)) * (NS * 128)
    rpt = NP // NS

    ei = jnp.concatenate(
        [edge_index.astype(jnp.int32),
         jnp.full((2, EP - E), N, jnp.int32)], axis=1)
    ei2 = ei.reshape(2, n_chunks, chunk)
    x_p = jnp.concatenate([x, jnp.zeros((NP - N, D), x.dtype)], axis=0)
    zrow = jnp.zeros((NP, D), jnp.float32)

    mesh = plsc.VectorSubcoreMesh(core_axis_name="c", subcore_axis_name="s")
    sc_params = pltpu.CompilerParams(needs_layout_passes=False)

    hist = pl.kernel(
        functools.partial(_degree_body, NP, ept),
        out_type=jax.ShapeDtypeStruct((NC, NS, NP), jnp.float32),
        mesh=mesh,
        compiler_params=sc_params,
        scratch_types=[
            pltpu.VMEM((ept,), jnp.int32),
            pltpu.VMEM((NP,), jnp.float32),
        ],
    )(ei)
    h0 = hist[0]
    h1 = hist[1]

    RB = rpt
    grid = NP // RB
    z = pl.pallas_call(
        _z_body,
        grid=(grid,),
        in_specs=[
            pl.BlockSpec((RB, D), lambda i: (i, 0)),
            pl.BlockSpec((D, D), lambda i: (0, 0)),
            pl.BlockSpec((NS, RB), lambda i: (0, i)),
        ],
        out_specs=pl.BlockSpec((RB, D), lambda i: (i, 0)),
        out_shape=jax.ShapeDtypeStruct((NP, D), jnp.float32),
    )(x_p, W_gcn, h0)

    sg = 32
    ts = pl.kernel(
        functools.partial(_gs_body, NP, cpt, sg, rpt),
        out_type=jax.ShapeDtypeStruct((NC, NP, D), jnp.float32),
        mesh=mesh,
        compiler_params=sc_params,
        scratch_types=[
            pltpu.VMEM((sg, chunk), jnp.int32),
            pltpu.VMEM((sg, chunk), jnp.int32),
            pltpu.VMEM((chunk, D), jnp.float32),
            pltpu.VMEM((chunk, D), jnp.float32),
            pltpu.VMEM_SHARED((NP, D), jnp.float32),
            pltpu.SemaphoreType.DMA,
            pltpu.SemaphoreType.DMA,
            pltpu.SemaphoreType.DMA,
            pltpu.SemaphoreType.DMA,
        ],
    )(ei2, z, x_p, zrow)

    out = pl.pallas_call(
        _final_body,
        grid=(grid,),
        in_specs=[
            pl.BlockSpec((RB, D), lambda i: (i, 0)),
            pl.BlockSpec((RB, D), lambda i: (i, 0)),
            pl.BlockSpec((RB, D), lambda i: (i, 0)),
            pl.BlockSpec((RB, D), lambda i: (i, 0)),
            pl.BlockSpec((NS, RB), lambda i: (0, i)),
            pl.BlockSpec((NS, RB), lambda i: (0, i)),
            pl.BlockSpec((D, D), lambda i: (0, 0)),
            pl.BlockSpec((1, D), lambda i: (0, 0)),
            pl.BlockSpec((1, D), lambda i: (0, 0)),
        ],
        out_specs=pl.BlockSpec((RB, D), lambda i: (i, 0)),
        out_shape=jax.ShapeDtypeStruct((NP, D), jnp.float32),
    )(x_p, ts[1], ts[0], z, h0, h1, W_diff,
      b_gcn.reshape(1, D), b_diff.reshape(1, D))

    return out[:N]


# speedup vs baseline: 13.2439x; 1.1671x over previous
import functools

import jax
import jax.numpy as jnp
from jax import lax
from jax.experimental import pallas as pl
from jax.experimental.pallas import tpu as pltpu
from jax.experimental.pallas import tpu_sc as plsc

NC = 2
NS = 16
L = 16
K = 4


def _degree_body(np_, ept, ei_ref, hist_ref, idx_v, hist_v):
    c = lax.axis_index("c")
    s = lax.axis_index("s")
    row = 1 - c
    pltpu.sync_copy(ei_ref.at[row, pl.ds(s * ept, ept)], idx_v)
    zeros = jnp.zeros((L,), jnp.float32)
    ones = jnp.ones((L,), jnp.float32)

    def zero_body(i, _):
        hist_v[pl.ds(i * L, L)] = zeros
        return 0

    lax.fori_loop(0, np_ // L, zero_body, 0)

    def count_body(i, _):
        iv = idx_v[pl.ds(i * L, L)]
        plsc.addupdate_scatter(hist_v, [iv], ones)
        return 0

    lax.fori_loop(0, ept // L, count_body, 0)
    pltpu.sync_copy(hist_v, hist_ref.at[c, s])


def _gs_body(cpt, sg, rpt, ei2_ref, z_ref, x_ref, zrow_ref, ts_ref,
             idx_g, idx_s, b0, b1, b2, b3, acc,
             g0, g1, g2, g3, s0, s1, s2, s3):
    bufs = (b0, b1, b2, b3)
    gsem = (g0, g1, g2, g3)
    ssem = (s0, s1, s2, s3)
    c = lax.axis_index("c")
    s = lax.axis_index("s")
    pltpu.sync_copy(zrow_ref.at[pl.ds(s * rpt, rpt)], acc.at[pl.ds(s * rpt, rpt)])
    plsc.subcore_barrier()

    def run(table_ref):
        def wait_g(b):
            pltpu.make_async_copy(table_ref.at[idx_g.at[0]], bufs[b], gsem[b]).wait()

        def wait_s(b):
            pltpu.make_async_copy(bufs[b], acc.at[idx_s.at[0]], ssem[b]).wait()

        def stage(st, _):
            base = s * cpt + st * sg
            pltpu.sync_copy(ei2_ref.at[c, pl.ds(base, sg)], idx_g)
            pltpu.sync_copy(ei2_ref.at[1 - c, pl.ds(base, sg)], idx_s)
            for b in range(K):
                pltpu.async_copy(table_ref.at[idx_g.at[b]], bufs[b], gsem[b])

            def group(i, _):
                jj = i * K
                for b in range(K):
                    wait_g(b)
                    pltpu.async_copy(bufs[b], acc.at[idx_s.at[jj + b]],
                                     ssem[b], add=True)
                for b in range(K):
                    wait_s(b)
                    pltpu.async_copy(table_ref.at[idx_g.at[jj + K + b]],
                                     bufs[b], gsem[b])
                return 0

            lax.fori_loop(0, sg // K - 1, group, 0)
            jj = sg - K
            for b in range(K):
                wait_g(b)
                pltpu.async_copy(bufs[b], acc.at[idx_s.at[jj + b]],
                                 ssem[b], add=True)
            for b in range(K):
                wait_s(b)
            return 0

        lax.fori_loop(0, cpt // sg, stage, 0)

    @pl.when(c == 0)
    def _():
        run(z_ref)

    @pl.when(c == 1)
    def _():
        run(x_ref)

    plsc.subcore_barrier()
    pltpu.sync_copy(acc.at[pl.ds(s * rpt, rpt)], ts_ref.at[c, pl.ds(s * rpt, rpt)])


def _z_body(x_ref, w_ref, h0_ref, z_ref):
    indeg = jnp.sum(h0_ref[...], axis=0)
    dis = lax.rsqrt(indeg + 1.0)
    xw = jnp.dot(x_ref[...], w_ref[...], preferred_element_type=jnp.float32)
    z_ref[...] = xw * dis[:, None]


def _final_body(x_ref, s_ref, t_ref, z_ref, h0_ref, h1_ref, wd_ref,
                bg_ref, bd_ref, o_ref):
    indeg = jnp.sum(h0_ref[...], axis=0)
    cnt = jnp.sum(h1_ref[...], axis=0)
    dis = lax.rsqrt(indeg + 1.0)
    h = jnp.maximum(dis[:, None] * (t_ref[...] + z_ref[...]) + bg_ref[...], 0.0)
    proj = jnp.dot(x_ref[...] * s_ref[...], wd_ref[...],
                   preferred_element_type=jnp.float32)
    o_ref[...] = h + proj + cnt[:, None] * bd_ref[...]


def kernel(x, edge_index, W_gcn, b_gcn, W_diff, b_diff):
    N, D = x.shape
    E = edge_index.shape[1]

    chunk = 64
    sg = 64
    n_chunks = -(-E // (NS * sg * chunk)) * (NS * sg)
    EP = n_chunks * chunk
    cpt = n_chunks // NS
    ept = EP // NS
    NP = -(-(N + 1) // 128) * 128
    rpt = NP // NS

    ei = jnp.concatenate(
        [edge_index.astype(jnp.int32),
         jnp.full((2, EP - E), N, jnp.int32)], axis=1)
    ei2 = ei.reshape(2, n_chunks, chunk)
    x_p = jnp.concatenate([x, jnp.zeros((NP - N, D), x.dtype)], axis=0)
    zrow = jnp.zeros((NP, D), jnp.float32)

    mesh = plsc.VectorSubcoreMesh(core_axis_name="c", subcore_axis_name="s")
    sc_params = pltpu.CompilerParams(needs_layout_passes=False)

    hist = pl.kernel(
        functools.partial(_degree_body, NP, ept),
        out_type=jax.ShapeDtypeStruct((NC, NS, NP), jnp.float32),
        mesh=mesh,
        compiler_params=sc_params,
        scratch_types=[
            pltpu.VMEM((ept,), jnp.int32),
            pltpu.VMEM((NP,), jnp.float32),
        ],
    )(ei)
    h0 = hist[0]
    h1 = hist[1]

    z = pl.pallas_call(
        _z_body,
        out_shape=jax.ShapeDtypeStruct((NP, D), jnp.float32),
    )(x_p, W_gcn, h0)

    ts = pl.kernel(
        functools.partial(_gs_body, cpt, sg, rpt),
        out_type=jax.ShapeDtypeStruct((NC, NP, D), jnp.float32),
        mesh=mesh,
        compiler_params=sc_params,
        scratch_types=[
            pltpu.VMEM((sg, chunk), jnp.int32),
            pltpu.VMEM((sg, chunk), jnp.int32),
            pltpu.VMEM((chunk, D), jnp.float32),
            pltpu.VMEM((chunk, D), jnp.float32),
            pltpu.VMEM((chunk, D), jnp.float32),
            pltpu.VMEM((chunk, D), jnp.float32),
            pltpu.VMEM_SHARED((NP, D), jnp.float32),
            pltpu.SemaphoreType.DMA,
            pltpu.SemaphoreType.DMA,
            pltpu.SemaphoreType.DMA,
            pltpu.SemaphoreType.DMA,
            pltpu.SemaphoreType.DMA,
            pltpu.SemaphoreType.DMA,
            pltpu.SemaphoreType.DMA,
            pltpu.SemaphoreType.DMA,
        ],
    )(ei2, z, x_p, zrow)

    out = pl.pallas_call(
        _final_body,
        out_shape=jax.ShapeDtypeStruct((NP, D), jnp.float32),
    )(x_p, ts[1], ts[0], z, h0, h1, W_diff,
      b_gcn.reshape(1, D), b_diff.reshape(1, D))

    return out[:N]


# speedup vs baseline: 13.2747x; 1.0023x over previous
import functools

import jax
import jax.numpy as jnp
from jax import lax
from jax.experimental import pallas as pl
from jax.experimental.pallas import tpu as pltpu
from jax.experimental.pallas import tpu_sc as plsc

NC = 2
NS = 16
L = 16
K = 4


def _degree_body(np_, ept, ei_ref, hist_ref, idx_v, hist_v):
    c = lax.axis_index("c")
    s = lax.axis_index("s")
    row = 1 - c
    pltpu.sync_copy(ei_ref.at[row, pl.ds(s * ept, ept)], idx_v)
    zeros = jnp.zeros((L,), jnp.float32)
    ones = jnp.ones((L,), jnp.float32)

    def zero_body(i, _):
        hist_v[pl.ds(i * L, L)] = zeros
        return 0

    lax.fori_loop(0, np_ // L, zero_body, 0)

    def count_body(i, _):
        for u in range(4):
            iv = idx_v[pl.ds((i * 4 + u) * L, L)]
            plsc.addupdate_scatter(hist_v, [iv], ones)
        return 0

    lax.fori_loop(0, ept // (L * 4), count_body, 0)
    pltpu.sync_copy(hist_v, hist_ref.at[c, s])


def _gs_body(cpt, sg, rpt, ei2_ref, z_ref, x_ref, zrow_ref, ts_ref,
             idx_g, idx_s, b0, b1, b2, b3, acc,
             g0, g1, g2, g3, s0, s1, s2, s3, isem_g, isem_s):
    bufs = (b0, b1, b2, b3)
    gsem = (g0, g1, g2, g3)
    ssem = (s0, s1, s2, s3)
    c = lax.axis_index("c")
    s = lax.axis_index("s")
    pltpu.sync_copy(zrow_ref.at[pl.ds(s * rpt, rpt)], acc.at[pl.ds(s * rpt, rpt)])
    plsc.subcore_barrier()

    nstages = cpt // sg

    def run(table_ref):
        def wait_g(b):
            pltpu.make_async_copy(table_ref.at[idx_g.at[0, 0]], bufs[b], gsem[b]).wait()

        def wait_s(b):
            pltpu.make_async_copy(bufs[b], acc.at[idx_s.at[0, 0]], ssem[b]).wait()

        def load_idx(slot, st):
            base = s * cpt + st * sg
            pltpu.async_copy(ei2_ref.at[c, pl.ds(base, sg)], idx_g.at[slot], isem_g)
            pltpu.async_copy(ei2_ref.at[1 - c, pl.ds(base, sg)], idx_s.at[slot], isem_s)

        def wait_idx():
            pltpu.make_async_copy(ei2_ref.at[c, pl.ds(0, sg)], idx_g.at[0], isem_g).wait()
            pltpu.make_async_copy(ei2_ref.at[1 - c, pl.ds(0, sg)], idx_s.at[0], isem_s).wait()

        load_idx(0, 0)

        def stage(st, _):
            slot = lax.rem(st, 2)
            wait_idx()

            @pl.when(st + 1 < nstages)
            def _():
                load_idx(1 - slot, st + 1)

            for b in range(K):
                pltpu.async_copy(table_ref.at[idx_g.at[slot, b]], bufs[b], gsem[b])

            def group(i, _):
                jj = i * K
                for b in range(K):
                    wait_g(b)
                    pltpu.async_copy(bufs[b], acc.at[idx_s.at[slot, jj + b]],
                                     ssem[b], add=True)
                for b in range(K):
                    wait_s(b)
                    pltpu.async_copy(table_ref.at[idx_g.at[slot, jj + K + b]],
                                     bufs[b], gsem[b])
                return 0

            lax.fori_loop(0, sg // K - 1, group, 0)
            jj = sg - K
            for b in range(K):
                wait_g(b)
                pltpu.async_copy(bufs[b], acc.at[idx_s.at[slot, jj + b]],
                                 ssem[b], add=True)
            for b in range(K):
                wait_s(b)
            return 0

        lax.fori_loop(0, nstages, stage, 0)

    @pl.when(c == 0)
    def _():
        run(z_ref)

    @pl.when(c == 1)
    def _():
        run(x_ref)

    plsc.subcore_barrier()
    pltpu.sync_copy(acc.at[pl.ds(s * rpt, rpt)], ts_ref.at[c, pl.ds(s * rpt, rpt)])


def _z_body(x_ref, w_ref, h0_ref, z_ref):
    indeg = jnp.sum(h0_ref[...], axis=0)
    dis = lax.rsqrt(indeg + 1.0)
    xw = jnp.dot(x_ref[...], w_ref[...], preferred_element_type=jnp.float32)
    z_ref[...] = xw * dis[:, None]


def _final_body(x_ref, s_ref, t_ref, z_ref, h0_ref, h1_ref, wd_ref,
                bg_ref, bd_ref, o_ref):
    indeg = jnp.sum(h0_ref[...], axis=0)
    cnt = jnp.sum(h1_ref[...], axis=0)
    dis = lax.rsqrt(indeg + 1.0)
    h = jnp.maximum(dis[:, None] * (t_ref[...] + z_ref[...]) + bg_ref[...], 0.0)
    proj = jnp.dot(x_ref[...] * s_ref[...], wd_ref[...],
                   preferred_element_type=jnp.float32)
    o_ref[...] = h + proj + cnt[:, None] * bd_ref[...]


def kernel(x, edge_index, W_gcn, b_gcn, W_diff, b_diff):
    N, D = x.shape
    E = edge_index.shape[1]

    chunk = 64
    sg = 32
    n_chunks = -(-E // (NS * sg * chunk)) * (NS * sg)
    EP = n_chunks * chunk
    cpt = n_chunks // NS
    ept = EP // NS
    NP = -(-(N + 1) // 128) * 128
    rpt = NP // NS

    ei = jnp.concatenate(
        [edge_index.astype(jnp.int32),
         jnp.full((2, EP - E), N, jnp.int32)], axis=1)
    ei2 = ei.reshape(2, n_chunks, chunk)
    x_p = jnp.concatenate([x, jnp.zeros((NP - N, D), x.dtype)], axis=0)
    zrow = jnp.zeros((NP, D), jnp.float32)

    mesh = plsc.VectorSubcoreMesh(core_axis_name="c", subcore_axis_name="s")
    sc_params = pltpu.CompilerParams(needs_layout_passes=False)

    hist = pl.kernel(
        functools.partial(_degree_body, NP, ept),
        out_type=jax.ShapeDtypeStruct((NC, NS, NP), jnp.float32),
        mesh=mesh,
        compiler_params=sc_params,
        scratch_types=[
            pltpu.VMEM((ept,), jnp.int32),
            pltpu.VMEM((NP,), jnp.float32),
        ],
    )(ei)
    h0 = hist[0]
    h1 = hist[1]

    z = pl.pallas_call(
        _z_body,
        out_shape=jax.ShapeDtypeStruct((NP, D), jnp.float32),
    )(x_p, W_gcn, h0)

    ts = pl.kernel(
        functools.partial(_gs_body, cpt, sg, rpt),
        out_type=jax.ShapeDtypeStruct((NC, NP, D), jnp.float32),
        mesh=mesh,
        compiler_params=sc_params,
        scratch_types=[
            pltpu.VMEM((2, sg, chunk), jnp.int32),
            pltpu.VMEM((2, sg, chunk), jnp.int32),
            pltpu.VMEM((chunk, D), jnp.float32),
            pltpu.VMEM((chunk, D), jnp.float32),
            pltpu.VMEM((chunk, D), jnp.float32),
            pltpu.VMEM((chunk, D), jnp.float32),
            pltpu.VMEM_SHARED((NP, D), jnp.float32),
            pltpu.SemaphoreType.DMA,
            pltpu.SemaphoreType.DMA,
            pltpu.SemaphoreType.DMA,
            pltpu.SemaphoreType.DMA,
            pltpu.SemaphoreType.DMA,
            pltpu.SemaphoreType.DMA,
            pltpu.SemaphoreType.DMA,
            pltpu.SemaphoreType.DMA,
            pltpu.SemaphoreType.DMA,
            pltpu.SemaphoreType.DMA,
        ],
    )(ei2, z, x_p, zrow)

    out = pl.pallas_call(
        _final_body,
        out_shape=jax.ShapeDtypeStruct((NP, D), jnp.float32),
    )(x_p, ts[1], ts[0], z, h0, h1, W_diff,
      b_gcn.reshape(1, D), b_diff.reshape(1, D))

    return out[:N]


# speedup vs baseline: 15.5302x; 1.1699x over previous
import functools

import jax
import jax.numpy as jnp
from jax import lax
from jax.experimental import pallas as pl
from jax.experimental.pallas import tpu as pltpu
from jax.experimental.pallas import tpu_sc as plsc

NC = 2
NS = 16
L = 16
K = 8


def _degree_body(np_, ept, ei_ref, hist_ref, idx_v, hist_v):
    c = lax.axis_index("c")
    s = lax.axis_index("s")
    row = 1 - c
    pltpu.sync_copy(ei_ref.at[row, pl.ds(s * ept, ept)], idx_v)
    zeros = jnp.zeros((L,), jnp.float32)
    ones = jnp.ones((L,), jnp.float32)

    def zero_body(i, _):
        hist_v[pl.ds(i * L, L)] = zeros
        return 0

    lax.fori_loop(0, np_ // L, zero_body, 0)

    def count_body(i, _):
        for u in range(4):
            iv = idx_v[pl.ds((i * 4 + u) * L, L)]
            plsc.addupdate_scatter(hist_v, [iv], ones)
        return 0

    lax.fori_loop(0, ept // (L * 4), count_body, 0)
    pltpu.sync_copy(hist_v, hist_ref.at[c, s])


def _gs_body(cpt, sg, rpt, ei2_ref, z_ref, x_ref, zrow_ref, ts_ref,
             idx_g, idx_s, b0, b1, b2, b3, b4, b5, b6, b7, acc,
             g0, g1, g2, g3, g4, g5, g6, g7,
             s0, s1, s2, s3, s4, s5, s6, s7, isem_g, isem_s):
    bufs = (b0, b1, b2, b3, b4, b5, b6, b7)
    gsem = (g0, g1, g2, g3, g4, g5, g6, g7)
    ssem = (s0, s1, s2, s3, s4, s5, s6, s7)
    c = lax.axis_index("c")
    s = lax.axis_index("s")
    pltpu.sync_copy(zrow_ref.at[pl.ds(s * rpt, rpt)], acc.at[pl.ds(s * rpt, rpt)])
    plsc.subcore_barrier()

    nstages = cpt // sg

    def run(table_ref):
        def wait_g(b):
            pltpu.make_async_copy(table_ref.at[idx_g.at[0, 0]], bufs[b], gsem[b]).wait()

        def wait_s(b):
            pltpu.make_async_copy(bufs[b], acc.at[idx_s.at[0, 0]], ssem[b]).wait()

        def load_idx(slot, st):
            base = s * cpt + st * sg
            pltpu.async_copy(ei2_ref.at[c, pl.ds(base, sg)], idx_g.at[slot], isem_g)
            pltpu.async_copy(ei2_ref.at[1 - c, pl.ds(base, sg)], idx_s.at[slot], isem_s)

        def wait_idx():
            pltpu.make_async_copy(ei2_ref.at[c, pl.ds(0, sg)], idx_g.at[0], isem_g).wait()
            pltpu.make_async_copy(ei2_ref.at[1 - c, pl.ds(0, sg)], idx_s.at[0], isem_s).wait()

        load_idx(0, 0)

        def stage(st, _):
            slot = lax.rem(st, 2)
            wait_idx()

            @pl.when(st + 1 < nstages)
            def _():
                load_idx(1 - slot, st + 1)

            for b in range(K):
                pltpu.async_copy(table_ref.at[idx_g.at[slot, b]], bufs[b], gsem[b])

            def group(i, _):
                jj = i * K
                for b in range(K):
                    wait_g(b)
                    pltpu.async_copy(bufs[b], acc.at[idx_s.at[slot, jj + b]],
                                     ssem[b], add=True)
                for b in range(K):
                    wait_s(b)
                    pltpu.async_copy(table_ref.at[idx_g.at[slot, jj + K + b]],
                                     bufs[b], gsem[b])
                return 0

            lax.fori_loop(0, sg // K - 1, group, 0)
            jj = sg - K
            for b in range(K):
                wait_g(b)
                pltpu.async_copy(bufs[b], acc.at[idx_s.at[slot, jj + b]],
                                 ssem[b], add=True)
            for b in range(K):
                wait_s(b)
            return 0

        lax.fori_loop(0, nstages, stage, 0)

    @pl.when(c == 0)
    def _():
        run(z_ref)

    @pl.when(c == 1)
    def _():
        run(x_ref)

    plsc.subcore_barrier()
    pltpu.sync_copy(acc.at[pl.ds(s * rpt, rpt)], ts_ref.at[c, pl.ds(s * rpt, rpt)])


def _z_body(x_ref, w_ref, h0_ref, z_ref):
    indeg = jnp.sum(h0_ref[...], axis=0)
    dis = lax.rsqrt(indeg + 1.0)
    xw = jnp.dot(x_ref[...], w_ref[...], preferred_element_type=jnp.float32)
    z_ref[...] = xw * dis[:, None]


def _final_body(x_ref, s_ref, t_ref, z_ref, h0_ref, h1_ref, wd_ref,
                bg_ref, bd_ref, o_ref):
    indeg = jnp.sum(h0_ref[...], axis=0)
    cnt = jnp.sum(h1_ref[...], axis=0)
    dis = lax.rsqrt(indeg + 1.0)
    h = jnp.maximum(dis[:, None] * (t_ref[...] + z_ref[...]) + bg_ref[...], 0.0)
    proj = jnp.dot(x_ref[...] * s_ref[...], wd_ref[...],
                   preferred_element_type=jnp.float32)
    o_ref[...] = h + proj + cnt[:, None] * bd_ref[...]


def kernel(x, edge_index, W_gcn, b_gcn, W_diff, b_diff):
    N, D = x.shape
    E = edge_index.shape[1]

    chunk = 32
    sg = 32
    n_chunks = -(-E // (NS * sg * chunk)) * (NS * sg)
    EP = n_chunks * chunk
    cpt = n_chunks // NS
    ept = EP // NS
    NP = -(-(N + 1) // 128) * 128
    rpt = NP // NS

    ei = jnp.concatenate(
        [edge_index.astype(jnp.int32),
         jnp.full((2, EP - E), N, jnp.int32)], axis=1)
    ei2 = ei.reshape(2, n_chunks, chunk)
    x_p = jnp.concatenate([x, jnp.zeros((NP - N, D), x.dtype)], axis=0)
    zrow = jnp.zeros((NP, D), jnp.float32)

    mesh = plsc.VectorSubcoreMesh(core_axis_name="c", subcore_axis_name="s")
    sc_params = pltpu.CompilerParams(needs_layout_passes=False)

    hist = pl.kernel(
        functools.partial(_degree_body, NP, ept),
        out_type=jax.ShapeDtypeStruct((NC, NS, NP), jnp.float32),
        mesh=mesh,
        compiler_params=sc_params,
        scratch_types=[
            pltpu.VMEM((ept,), jnp.int32),
            pltpu.VMEM((NP,), jnp.float32),
        ],
    )(ei)
    h0 = hist[0]
    h1 = hist[1]

    z = pl.pallas_call(
        _z_body,
        out_shape=jax.ShapeDtypeStruct((NP, D), jnp.float32),
    )(x_p, W_gcn, h0)

    ts = pl.kernel(
        functools.partial(_gs_body, cpt, sg, rpt),
        out_type=jax.ShapeDtypeStruct((NC, NP, D), jnp.float32),
        mesh=mesh,
        compiler_params=sc_params,
        scratch_types=[
            pltpu.VMEM((2, sg, chunk), jnp.int32),
            pltpu.VMEM((2, sg, chunk), jnp.int32),
            pltpu.VMEM((chunk, D), jnp.float32),
            pltpu.VMEM((chunk, D), jnp.float32),
            pltpu.VMEM((chunk, D), jnp.float32),
            pltpu.VMEM((chunk, D), jnp.float32),
            pltpu.VMEM((chunk, D), jnp.float32),
            pltpu.VMEM((chunk, D), jnp.float32),
            pltpu.VMEM((chunk, D), jnp.float32),
            pltpu.VMEM((chunk, D), jnp.float32),
            pltpu.VMEM_SHARED((NP, D), jnp.float32),
            pltpu.SemaphoreType.DMA,
            pltpu.SemaphoreType.DMA,
            pltpu.SemaphoreType.DMA,
            pltpu.SemaphoreType.DMA,
            pltpu.SemaphoreType.DMA,
            pltpu.SemaphoreType.DMA,
            pltpu.SemaphoreType.DMA,
            pltpu.SemaphoreType.DMA,
            pltpu.SemaphoreType.DMA,
            pltpu.SemaphoreType.DMA,
            pltpu.SemaphoreType.DMA,
            pltpu.SemaphoreType.DMA,
            pltpu.SemaphoreType.DMA,
            pltpu.SemaphoreType.DMA,
            pltpu.SemaphoreType.DMA,
            pltpu.SemaphoreType.DMA,
            pltpu.SemaphoreType.DMA,
            pltpu.SemaphoreType.DMA,
        ],
    )(ei2, z, x_p, zrow)

    out = pl.pallas_call(
        _final_body,
        out_shape=jax.ShapeDtypeStruct((NP, D), jnp.float32),
    )(x_p, ts[1], ts[0], z, h0, h1, W_diff,
      b_gcn.reshape(1, D), b_diff.reshape(1, D))

    return out[:N]
